# baseline (speedup 1.0000x reference)
import jax
import jax.numpy as jnp
from jax.experimental import pallas as pl
from jax.experimental.pallas import tpu as pltpu

_B, _D, _E, _H, _C = 2048, 1024, 16, 256, 100
_CP = 128


def _fused_kernel(x_ref, proto_ref, W1_ref, b1_ref, g_ref, bb_ref,
                  W2_ref, b2_ref, out_ref, w_scratch, x16_scratch, h_scratch):
    e = pl.program_id(0)

    @pl.when(e == 0)
    def _compute_router():
        xf = x_ref[...]
        xn = xf / (jnp.sqrt(jnp.sum(xf * xf, axis=1, keepdims=True)) + 1e-8)
        p = proto_ref[...]
        pn = p / (jnp.sqrt(jnp.sum(p * p, axis=1, keepdims=True)) + 1e-8)
        sims = jnp.dot(xn, pn.T, preferred_element_type=jnp.float32)
        w_scratch[...] = jax.nn.softmax(sims, axis=-1)
        x16_scratch[...] = xf.astype(jnp.bfloat16)

    w1 = W1_ref[0].astype(jnp.bfloat16)
    xb = x16_scratch[...]
    h_new = jnp.dot(xb, w1, preferred_element_type=jnp.float32) + b1_ref[0]
    h_scratch[e % 2] = h_new

    ep = e - 1
    h = h_scratch[(e + 1) % 2]
    mu = jnp.mean(h, axis=-1, keepdims=True)
    var = jnp.mean(h * h, axis=-1, keepdims=True) - mu * mu
    rstd = jax.lax.rsqrt(var + 1e-5)
    hn = h * rstd - mu * rstd
    hg = hn * g_ref[0] + bb_ref[0]
    hgelu = jax.nn.gelu(hg).astype(jnp.bfloat16)
    w2 = W2_ref[0].astype(jnp.bfloat16)
    logits = (jnp.dot(hgelu, w2, preferred_element_type=jnp.float32)
              + b2_ref[0])

    w = w_scratch[...]
    lane = jax.lax.broadcasted_iota(jnp.int32, w.shape, 1)
    w_col = jnp.sum(jnp.where(lane == ep, w, 0.0), axis=1, keepdims=True)
    acc = w_col * logits
    prev = jnp.where(e >= 2, out_ref[...], 0.0)
    out_ref[...] = prev + acc


@jax.jit
def kernel(x, prototypes, W1, b1, ln_g, ln_b, W2, b2):
    W2p = jnp.pad(W2, ((0, 0), (0, 0), (0, _CP - _C)))
    b2p = jnp.pad(b2, ((0, 0), (0, _CP - _C)))
    b1r = b1.reshape(_E, 1, _H)
    gr = ln_g.reshape(_E, 1, _H)
    br = ln_b.reshape(_E, 1, _H)
    b2r = b2p.reshape(_E, 1, _CP)

    def _prod_ix(e):
        i = jnp.minimum(e, _E - 1)
        return (i, 0, 0)

    def _cons_ix(e):
        i = jnp.maximum(e - 1, 0)
        return (i, 0, 0)

    out = pl.pallas_call(
        _fused_kernel,
        grid=(_E + 1,),
        in_specs=[
            pl.BlockSpec((_B, _D), lambda e: (0, 0)),
            pl.BlockSpec((_E, _D), lambda e: (0, 0)),
            pl.BlockSpec((1, _D, _H), _prod_ix),
            pl.BlockSpec((1, 1, _H), _prod_ix),
            pl.BlockSpec((1, 1, _H), _cons_ix),
            pl.BlockSpec((1, 1, _H), _cons_ix),
            pl.BlockSpec((1, _H, _CP), _cons_ix),
            pl.BlockSpec((1, 1, _CP), _cons_ix),
        ],
        out_specs=pl.BlockSpec((_B, _CP), lambda e: (0, 0)),
        out_shape=jax.ShapeDtypeStruct((_B, _CP), jnp.float32),
        scratch_shapes=[pltpu.VMEM((_B, _E), jnp.float32),
                        pltpu.VMEM((_B, _D), jnp.bfloat16),
                        pltpu.VMEM((2, _B, _H), jnp.float32)],
        compiler_params=pltpu.CompilerParams(
            dimension_semantics=("arbitrary",)),
    )(x, prototypes, W1, b1r, gr, br, W2p, b2r)
    return out[:, :_C]


# speedup vs baseline: 1.2786x; 1.2786x over previous
import jax
import jax.numpy as jnp
from jax.experimental import pallas as pl
from jax.experimental.pallas import tpu as pltpu

_B, _D, _E, _H, _C = 2048, 1024, 16, 256, 100
_CP = 128


def _fused_kernel(x_ref, proto_ref, W1p_ref, b1p_ref, W1a_ref, b1a_ref,
                  W1b_ref, b1b_ref, g1_ref, bb1_ref, W21_ref, b21_ref,
                  g2_ref, bb2_ref, W22_ref, b22_ref, out_ref,
                  w_scratch, x16_scratch, hA, hB):
    k = pl.program_id(0)

    @pl.when(k == 0)
    def _prologue():
        xf = x_ref[...]
        xn = xf / (jnp.sqrt(jnp.sum(xf * xf, axis=1, keepdims=True)) + 1e-8)
        p = proto_ref[...]
        pn = p / (jnp.sqrt(jnp.sum(p * p, axis=1, keepdims=True)) + 1e-8)
        sims = jnp.dot(xn, pn.T, preferred_element_type=jnp.float32)
        w_scratch[...] = jax.nn.softmax(sims, axis=-1)
        x16 = xf.astype(jnp.bfloat16)
        x16_scratch[...] = x16
        w1 = W1p_ref[0].astype(jnp.bfloat16)
        hA[...] = (jnp.dot(x16, w1, preferred_element_type=jnp.float32)
                   + b1p_ref[0])

    xb = x16_scratch[...]
    w = w_scratch[...]
    lane = jax.lax.broadcasted_iota(jnp.int32, w.shape, 1)

    def consume(h, g, bb, w2_ref, b2_ref, ei):
        mu = jnp.mean(h, axis=-1, keepdims=True)
        var = jnp.mean(h * h, axis=-1, keepdims=True) - mu * mu
        rstd = jax.lax.rsqrt(var + 1e-5)
        hn = h * rstd - mu * rstd
        hg = hn * g + bb
        hgelu = jax.nn.gelu(hg).astype(jnp.bfloat16)
        w2 = w2_ref[0].astype(jnp.bfloat16)
        logits = (jnp.dot(hgelu, w2, preferred_element_type=jnp.float32)
                  + b2_ref[0])
        w_col = jnp.sum(jnp.where(lane == ei, w, 0.0), axis=1, keepdims=True)
        return w_col * logits

    h1 = (jnp.dot(xb, W1a_ref[0].astype(jnp.bfloat16),
                  preferred_element_type=jnp.float32) + b1a_ref[0])
    acc1 = consume(hA[...], g1_ref[0], bb1_ref[0], W21_ref, b21_ref, 2 * k)
    hB[...] = h1

    h2 = (jnp.dot(xb, W1b_ref[0].astype(jnp.bfloat16),
                  preferred_element_type=jnp.float32) + b1b_ref[0])
    acc2 = consume(hB[...], g2_ref[0], bb2_ref[0], W22_ref, b22_ref,
                   2 * k + 1)
    hA[...] = h2

    prev = jnp.where(k >= 1, out_ref[...], 0.0)
    out_ref[...] = prev + acc1 + acc2


def _pallas_call():
    def _zero_ix(k):
        return (0, 0, 0)

    def _p1_ix(k):
        return (2 * k + 1, 0, 0)

    def _p2_ix(k):
        return (jnp.minimum(2 * k + 2, _E - 1), 0, 0)

    def _c1_ix(k):
        return (2 * k, 0, 0)

    def _c2_ix(k):
        return (2 * k + 1, 0, 0)

    return pl.pallas_call(
        _fused_kernel,
        grid=(_E // 2,),
        in_specs=[
            pl.BlockSpec((_B, _D), lambda k: (0, 0)),
            pl.BlockSpec((_E, _D), lambda k: (0, 0)),
            pl.BlockSpec((1, _D, _H), _zero_ix),
            pl.BlockSpec((1, 1, _H), _zero_ix),
            pl.BlockSpec((1, _D, _H), _p1_ix),
            pl.BlockSpec((1, 1, _H), _p1_ix),
            pl.BlockSpec((1, _D, _H), _p2_ix),
            pl.BlockSpec((1, 1, _H), _p2_ix),
            pl.BlockSpec((1, 1, _H), _c1_ix),
            pl.BlockSpec((1, 1, _H), _c1_ix),
            pl.BlockSpec((1, _H, _CP), _c1_ix),
            pl.BlockSpec((1, 1, _CP), _c1_ix),
            pl.BlockSpec((1, 1, _H), _c2_ix),
            pl.BlockSpec((1, 1, _H), _c2_ix),
            pl.BlockSpec((1, _H, _CP), _c2_ix),
            pl.BlockSpec((1, 1, _CP), _c2_ix),
        ],
        out_specs=pl.BlockSpec((_B, _CP), lambda k: (0, 0)),
        out_shape=jax.ShapeDtypeStruct((_B, _CP), jnp.float32),
        scratch_shapes=[pltpu.VMEM((_B, _E), jnp.float32),
                        pltpu.VMEM((_B, _D), jnp.bfloat16),
                        pltpu.VMEM((_B, _H), jnp.float32),
                        pltpu.VMEM((_B, _H), jnp.float32)],
        compiler_params=pltpu.CompilerParams(
            dimension_semantics=("arbitrary",)),
    )


@jax.jit
def kernel(x, prototypes, W1, b1, ln_g, ln_b, W2, b2):
    W2p = jnp.pad(W2, ((0, 0), (0, 0), (0, _CP - _C)))
    b2p = jnp.pad(b2, ((0, 0), (0, _CP - _C)))
    b1r = b1.reshape(_E, 1, _H)
    gr = ln_g.reshape(_E, 1, _H)
    br = ln_b.reshape(_E, 1, _H)
    b2r = b2p.reshape(_E, 1, _CP)
    out = _pallas_call()(
        x, prototypes, W1, b1r, W1, b1r, W1, b1r,
        gr, br, W2p, b2r, gr, br, W2p, b2r)
    return out[:, :_C]
